# Initial kernel scaffold; baseline (speedup 1.0000x reference)
import jax
import jax.numpy as jnp
from jax.experimental import pallas as pl


def kernel(adj_indices, adj_values, embeds):
    raise NotImplementedError("write your pallas kernel here")


# speedup vs baseline: 4.5514x; 4.5514x over previous
import functools

import jax
import jax.numpy as jnp
from jax import lax
from jax.experimental import pallas as pl
from jax.experimental.pallas import tpu as pltpu
from jax.experimental.pallas import tpu_sc as plsc

N_NODES = 10000
N_EDGES = 320000
D_FEAT = 128
LANES = 16
NUM_CORES = 2
NUM_SUBCORES = 16
NUM_TILES = NUM_CORES * NUM_SUBCORES
EDGES_PER_TILE = N_EDGES // NUM_TILES
CHUNK = 80
CHUNKS_PER_TILE = EDGES_PER_TILE // CHUNK
ROWS_PER_TILE = 624
ZROWS = 208
SLOPE = 0.5


def _sc_body(row_hbm, col_hbm, val_hbm, emb_hbm, out_hbm,
             colv, rowv, valv, rowsb, zbuf, shared, sem):
    c = lax.axis_index("c")
    s = lax.axis_index("s")
    wid = c * NUM_SUBCORES + s

    def _zero_z(i, _):
        for j in range(D_FEAT // LANES):
            zbuf[i, pl.ds(j * LANES, LANES)] = jnp.zeros((LANES,), jnp.float32)
        return 0
    lax.fori_loop(0, ZROWS, _zero_z, 0)
    row0 = s * ROWS_PER_TILE
    for t in range(ROWS_PER_TILE // ZROWS):
        pltpu.sync_copy(zbuf, shared.at[pl.ds(row0 + t * ZROWS, ZROWS)])
    @pl.when(s == NUM_SUBCORES - 1)
    def _zero_rem():
        pltpu.sync_copy(zbuf.at[pl.ds(0, N_NODES - NUM_SUBCORES * ROWS_PER_TILE)],
                        shared.at[pl.ds(NUM_SUBCORES * ROWS_PER_TILE,
                                        N_NODES - NUM_SUBCORES * ROWS_PER_TILE)])
    plsc.subcore_barrier()

    def _chunk(k, _):
        base = wid * EDGES_PER_TILE + k * CHUNK
        pltpu.sync_copy(col_hbm.at[pl.ds(base, CHUNK)], colv)
        pltpu.sync_copy(row_hbm.at[pl.ds(base, CHUNK)], rowv)
        pltpu.sync_copy(val_hbm.at[pl.ds(base, CHUNK)], valv)
        pltpu.async_copy(emb_hbm.at[colv], rowsb, sem).wait()

        def _scale(g, _):
            val16 = valv[pl.ds(g * LANES, LANES)]
            for e_loc in range(LANES):
                bvec = jnp.take_along_axis(
                    val16, jnp.full((LANES,), e_loc, jnp.int32), axis=0)
                e = g * LANES + e_loc
                for j in range(D_FEAT // LANES):
                    sl = pl.ds(j * LANES, LANES)
                    rowsb[e, sl] = rowsb[e, sl] * bvec
            return 0
        lax.fori_loop(0, CHUNK // LANES, _scale, 0)

        pltpu.sync_copy(rowsb, shared.at[rowv], add=True)
        return 0
    lax.fori_loop(0, CHUNKS_PER_TILE, _chunk, 0)
    plsc.subcore_barrier()

    pltpu.sync_copy(shared.at[pl.ds(row0, ROWS_PER_TILE)],
                    out_hbm.at[c, pl.ds(row0, ROWS_PER_TILE)])
    @pl.when(s == NUM_SUBCORES - 1)
    def _write_rem():
        r = NUM_SUBCORES * ROWS_PER_TILE
        pltpu.sync_copy(shared.at[pl.ds(r, N_NODES - r)],
                        out_hbm.at[c, pl.ds(r, N_NODES - r)])


@functools.partial(
    pl.kernel,
    out_type=jax.ShapeDtypeStruct((NUM_CORES, N_NODES, D_FEAT), jnp.float32),
    mesh=plsc.VectorSubcoreMesh(core_axis_name="c", subcore_axis_name="s"),
    scratch_types=[
        pltpu.VMEM((CHUNK,), jnp.int32),
        pltpu.VMEM((CHUNK,), jnp.int32),
        pltpu.VMEM((CHUNK,), jnp.float32),
        pltpu.VMEM((CHUNK, D_FEAT), jnp.float32),
        pltpu.VMEM((ZROWS, D_FEAT), jnp.float32),
        pltpu.VMEM_SHARED((N_NODES, D_FEAT), jnp.float32),
        pltpu.SemaphoreType.DMA,
    ],
)
def _sc_spmm(row_hbm, col_hbm, val_hbm, emb_hbm, out_hbm,
             colv, rowv, valv, rowsb, zbuf, shared, sem):
    _sc_body(row_hbm, col_hbm, val_hbm, emb_hbm, out_hbm,
             colv, rowv, valv, rowsb, zbuf, shared, sem)


def _combine_body(p_ref, o_ref):
    x = p_ref[0] + p_ref[1]
    o_ref[...] = jnp.where(x >= 0, x, SLOPE * x)


def _combine(partials):
    blk = 1000
    return pl.pallas_call(
        _combine_body,
        grid=(N_NODES // blk,),
        in_specs=[pl.BlockSpec((NUM_CORES, blk, D_FEAT), lambda i: (0, i, 0))],
        out_specs=pl.BlockSpec((blk, D_FEAT), lambda i: (i, 0)),
        out_shape=jax.ShapeDtypeStruct((N_NODES, D_FEAT), jnp.float32),
    )(partials)


def kernel(adj_indices, adj_values, embeds):
    idx = adj_indices.astype(jnp.int32)
    partials = _sc_spmm(idx[0], idx[1], adj_values, embeds)
    return _combine(partials)
